# baseline (speedup 1.0000x reference)
import functools

import jax
import jax.numpy as jnp
from jax import lax
from jax.experimental import pallas as pl
from jax.experimental.pallas import tpu as pltpu
from jax.experimental.pallas import tpu_sc as plsc

_NEG_SLOPE = 0.2
_EPS = 1e-9


def _tc_feat_body(h_ref, wf_ref, wer_ref, p_ref, featp_ref, er_ref):
    h = h_ref[...]
    featp_ref[...] = (
        jnp.dot(h, wf_ref[...], preferred_element_type=jnp.float32) + p_ref[...]
    )
    er_ref[...] = jnp.dot(h, wer_ref[...], preferred_element_type=jnp.float32)


def _tc_feat_call(h, wf, wer, p, block_rows=2000):
    n, k = h.shape
    wfw = wf.shape[1]
    return pl.pallas_call(
        _tc_feat_body,
        grid=(n // block_rows,),
        in_specs=[
            pl.BlockSpec((block_rows, k), lambda i: (i, 0)),
            pl.BlockSpec((k, wfw), lambda i: (0, 0)),
            pl.BlockSpec((k, 16), lambda i: (0, 0)),
            pl.BlockSpec((1, wfw), lambda i: (0, 0)),
        ],
        out_specs=[
            pl.BlockSpec((block_rows, wfw), lambda i: (i, 0)),
            pl.BlockSpec((block_rows, 16), lambda i: (i, 0)),
        ],
        out_shape=[
            jax.ShapeDtypeStruct((n, wfw), jnp.float32),
            jax.ShapeDtypeStruct((n, 16), jnp.float32),
        ],
    )(h, wf, wer, p)


def _tc_norm_feat_body(acc_ref, selw_ref, b_ref, wf_ref, wer_ref, p_ref,
                       featp_ref, er_ref):
    a = acc_ref[0] + acc_ref[1]
    den = a[:, 128:132]
    denb = jnp.dot(den, selw_ref[...], preferred_element_type=jnp.float32)
    h = a[:, :128] / (denb + _EPS) + b_ref[...]
    featp_ref[...] = (
        jnp.dot(h, wf_ref[...], preferred_element_type=jnp.float32) + p_ref[...]
    )
    er_ref[...] = jnp.dot(h, wer_ref[...], preferred_element_type=jnp.float32)


def _tc_norm_feat_call(acc, selw, brow, wf, wer, p, n, block_rows=2000):
    nc, _, wacc = acc.shape
    wfw = wf.shape[1]
    return pl.pallas_call(
        _tc_norm_feat_body,
        grid=(n // block_rows,),
        in_specs=[
            pl.BlockSpec((nc, block_rows, wacc), lambda i: (0, i, 0)),
            pl.BlockSpec((4, 128), lambda i: (0, 0)),
            pl.BlockSpec((1, 128), lambda i: (0, 0)),
            pl.BlockSpec((128, wfw), lambda i: (0, 0)),
            pl.BlockSpec((128, 16), lambda i: (0, 0)),
            pl.BlockSpec((1, wfw), lambda i: (0, 0)),
        ],
        out_specs=[
            pl.BlockSpec((block_rows, wfw), lambda i: (i, 0)),
            pl.BlockSpec((block_rows, 16), lambda i: (i, 0)),
        ],
        out_shape=[
            jax.ShapeDtypeStruct((n, wfw), jnp.float32),
            jax.ShapeDtypeStruct((n, 16), jnp.float32),
        ],
    )(acc, selw, brow, wf, wer, p)


def _tc_final_body(acc_ref, b_ref, out_ref):
    a = acc_ref[0] + acc_ref[1]
    den = a[:, 40:41]
    out_ref[...] = a[:, :40] / (den + _EPS) + b_ref[...]


def _tc_final_call(acc, brow, n, block_rows=2000):
    nc, _, wacc = acc.shape
    return pl.pallas_call(
        _tc_final_body,
        grid=(n // block_rows,),
        in_specs=[
            pl.BlockSpec((nc, block_rows, wacc), lambda i: (0, i, 0)),
            pl.BlockSpec((1, 40), lambda i: (0, 0)),
        ],
        out_specs=pl.BlockSpec((block_rows, 40), lambda i: (i, 0)),
        out_shape=jax.ShapeDtypeStruct((n, 40), jnp.float32),
    )(acc, brow)


@functools.lru_cache(maxsize=None)
def _make_sc_edge_kernel(n, e, wf, fcols, heads, b):
    nc, ns = 2, 16
    nw = nc * ns
    e_per_tile = e // nw
    iters = e_per_tile // b
    assert iters * b == e_per_tile and (iters - 2) % 3 == 0
    nv = wf // 16
    npad = ((n + 8 * ns - 1) // (8 * ns)) * (8 * ns)
    rows_per_tile = npad // ns
    sgroups = (b * heads) // 16
    mesh = plsc.VectorSubcoreMesh(
        core_axis_name="c", subcore_axis_name="s",
        num_cores=nc, num_subcores=ns)

    @functools.partial(
        pl.kernel,
        mesh=mesh,
        compiler_params=pltpu.CompilerParams(
            use_tc_tiling_on_sc=False, needs_layout_passes=False),
        out_type=jax.ShapeDtypeStruct((nc, npad, wf), jnp.float32),
        scratch_types=[
            pltpu.VMEM((3, b), jnp.int32),
            pltpu.VMEM((3, b), jnp.int32),
            pltpu.VMEM((3, b, wf), jnp.float32),
            pltpu.VMEM((3, b, 16), jnp.float32),
            pltpu.VMEM((b * heads + 16,), jnp.float32),
            pltpu.VMEM_SHARED((npad, wf), jnp.float32),
            [pltpu.SemaphoreType.DMA] * 3,
            [pltpu.SemaphoreType.DMA] * 3,
            [pltpu.SemaphoreType.DMA] * 3,
            [pltpu.SemaphoreType.DMA] * 3,
        ],
    )
    def sc_kernel(featp_hbm, er_hbm, src_hbm, dst_hbm, out_hbm,
                  srcv3, dstv3, fb3, eb3, sbuf, acc,
                  semi, semf, seme, semsc):
        cid = lax.axis_index("c")
        sid = lax.axis_index("s")
        wid = sid * nc + cid
        lane = lax.iota(jnp.int32, 16)
        zvec = jnp.zeros((16,), jnp.float32)
        last = iters - 1

        def zbody(r, _):
            for j in range(nv):
                fb3[0, r, pl.ds(j * 16, 16)] = zvec
            return 0

        lax.fori_loop(0, b, zbody, 0)
        row0 = sid * rows_per_tile
        zfull2 = rows_per_tile // b
        ztail2 = rows_per_tile - zfull2 * b
        assert ztail2 % 8 == 0
        for t in range(zfull2):
            pltpu.sync_copy(fb3.at[0], acc.at[pl.ds(row0 + t * b, b)])
        if ztail2:
            pltpu.sync_copy(
                fb3.at[0, pl.ds(0, ztail2)],
                acc.at[pl.ds(row0 + zfull2 * b, ztail2)])

        if heads == 4:
            s_rowp = lane >> 2
            s_colp_el = fcols + heads + (lane & 3)
            s_colp_er = lane & 3
        else:
            s_rowp = lane
            s_colp_el = jnp.full((16,), fcols + 1, jnp.int32)
            s_colp_er = jnp.zeros((16,), jnp.int32)

        def idx_start(k, bi):
            base = pl.multiple_of(wid * e_per_tile + k * b, 8)
            pltpu.async_copy(src_hbm.at[pl.ds(base, b)], srcv3.at[bi],
                             semi[bi])
            pltpu.async_copy(dst_hbm.at[pl.ds(base, b)], dstv3.at[bi],
                             semi[bi])

        def idx_wait(bi):
            pltpu.make_async_copy(
                src_hbm.at[pl.ds(0, b)], srcv3.at[bi], semi[bi]).wait()
            pltpu.make_async_copy(
                dst_hbm.at[pl.ds(0, b)], dstv3.at[bi], semi[bi]).wait()

        def gather_start(bi):
            pltpu.async_copy(featp_hbm.at[srcv3.at[bi]], fb3.at[bi],
                             semf[bi])
            pltpu.async_copy(er_hbm.at[dstv3.at[bi]], eb3.at[bi], seme[bi])

        def gather_wait(bi):
            pltpu.make_async_copy(
                featp_hbm.at[srcv3.at[bi]], fb3.at[bi], semf[bi]).wait()
            pltpu.make_async_copy(
                er_hbm.at[dstv3.at[bi]], eb3.at[bi], seme[bi]).wait()

        def scatter_start(bi):
            pltpu.async_copy(fb3.at[bi], acc.at[dstv3.at[bi]], semsc[bi],
                             add=True)

        def scatter_wait(bi):
            pltpu.make_async_copy(
                fb3.at[bi], acc.at[dstv3.at[bi]], semsc[bi]).wait()

        def compute(bi):
            fb = fb3.at[bi]
            eb = eb3.at[bi]

            def s_one(g):
                if heads == 4:
                    r = g * 4 + s_rowp
                else:
                    r = g * 16 + s_rowp
                el = plsc.load_gather(fb, [r, s_colp_el])
                er = plsc.load_gather(eb, [r, s_colp_er])
                v = el + er
                v = jnp.where(v >= 0.0, v, _NEG_SLOPE * v)
                sbuf[pl.ds(g * 16, 16)] = jnp.exp(v)

            def sbody(g, _):
                s_one(g)
                return 0

            lax.fori_loop(0, sgroups, sbody, 0)

            def scale_one(k):
                kh = k * heads
                if heads == 4:
                    svs = [
                        plsc.load_gather(
                            sbuf, [jnp.full((16,), kh + h, jnp.int32)])
                        for h in range(4)
                    ]
                    for j in range(nv - 1):
                        fb[k, pl.ds(j * 16, 16)] = (
                            fb[k, pl.ds(j * 16, 16)] * svs[j // 2])
                    squad = plsc.load_gather(sbuf, [kh + lane])
                    fb[k, pl.ds((nv - 1) * 16, 16)] = squad
                else:
                    sv = plsc.load_gather(
                        sbuf, [jnp.full((16,), k, jnp.int32)])
                    for j in range(nv - 1):
                        fb[k, pl.ds(j * 16, 16)] = (
                            fb[k, pl.ds(j * 16, 16)] * sv)
                    last_v = fb[k, pl.ds((nv - 1) * 16, 16)] * sv
                    fb[k, pl.ds((nv - 1) * 16, 16)] = jnp.where(
                        lane < 8, last_v, sv)

            def mbody(k, _):
                scale_one(k)
                return 0

            lax.fori_loop(0, b, mbody, 0)

        idx_start(0, 0)
        idx_start(1, 1)
        idx_start(2, 2)
        idx_wait(0)
        gather_start(0)
        idx_wait(1)
        gather_start(1)
        plsc.subcore_barrier()

        gather_wait(0)
        compute(0)
        idx_wait(2)
        gather_start(2)
        scatter_start(0)
        gather_wait(1)
        compute(1)
        scatter_wait(0)
        idx_start(3, 0)
        idx_wait(0)
        gather_start(0)
        scatter_start(1)

        def step(k, bi):
            gather_wait(bi)
            compute(bi)
            nb = (bi + 2) % 3
            scatter_wait(nb)
            idx_start(jnp.minimum(k + 2, last), nb)
            idx_wait(nb)
            gather_start(nb)
            scatter_start(bi)

        def pbody(g, _):
            k = 3 * g + 2
            step(k, 2)
            step(k + 1, 0)
            step(k + 2, 1)
            return 0

        lax.fori_loop(0, (iters - 2) // 3, pbody, 0)

        gather_wait(2)
        gather_wait(0)
        scatter_wait(1)

        plsc.subcore_barrier()
        pltpu.sync_copy(acc.at[pl.ds(row0, rows_per_tile)],
                        out_hbm.at[cid, pl.ds(row0, rows_per_tile)])

    return sc_kernel


def _prep_layer_weights(W, al, ar, heads, dout, wfw):
    k = W.shape[0]
    fcols = heads * dout
    wal = jnp.einsum("khd,hd->kh", W.reshape(k, heads, dout), al)
    war = jnp.einsum("khd,hd->kh", W.reshape(k, heads, dout), ar)
    wf = jnp.zeros((k, wfw), jnp.float32)
    wf = wf.at[:, :fcols].set(W)
    wf = wf.at[:, fcols + heads:fcols + 2 * heads].set(wal)
    wer = jnp.zeros((k, 16), jnp.float32)
    wer = wer.at[:, :heads].set(war)
    p = jnp.zeros((1, wfw), jnp.float32)
    p = p.at[0, fcols:fcols + heads].set(1.0)
    return wf, wer, p


def kernel(x, edge_index0, edge_index1, edge_index2,
           W0, al0, ar0, b0, W1, al1, ar1, b1, W2, al2, ar2, b2):
    n = x.shape[0]
    e = edge_index0.shape[1]

    wf0, wer0, p0 = _prep_layer_weights(W0, al0, ar0, 4, 32, 144)
    wf1, wer1, p1 = _prep_layer_weights(W1, al1, ar1, 4, 32, 144)
    wf2, wer2, p2 = _prep_layer_weights(W2, al2, ar2, 1, 40, 48)

    selw = jnp.zeros((4, 128), jnp.float32)
    for h in range(4):
        selw = selw.at[h, h * 32:(h + 1) * 32].set(1.0)
    b0row = b0.reshape(1, 128)
    b1row = b1.reshape(1, 128)
    b2row = b2.reshape(1, 40)

    bsz = 80
    ei0 = edge_index0.astype(jnp.int32)
    ei1 = edge_index1.astype(jnp.int32)
    ei2 = edge_index2.astype(jnp.int32)

    sc144 = _make_sc_edge_kernel(n, e, 144, 128, 4, bsz)
    sc48 = _make_sc_edge_kernel(n, e, 48, 40, 1, bsz)

    featp, er = _tc_feat_call(x, wf0, wer0, p0)
    acc0 = sc144(featp, er, ei0[0], ei0[1])

    featp, er = _tc_norm_feat_call(acc0, selw, b0row, wf1, wer1, p1, n)
    acc1 = sc144(featp, er, ei1[0], ei1[1])

    featp, er = _tc_norm_feat_call(acc1, selw, b1row, wf2, wer2, p2, n)
    acc2 = sc48(featp, er, ei2[0], ei2[1])

    return _tc_final_call(acc2, b2row, n)


# speedup vs baseline: 1.0718x; 1.0718x over previous
import functools

import jax
import jax.numpy as jnp
from jax import lax
from jax.experimental import pallas as pl
from jax.experimental.pallas import tpu as pltpu
from jax.experimental.pallas import tpu_sc as plsc

_NEG_SLOPE = 0.2
_EPS = 1e-9


def _tc_feat_body(h_ref, wf_ref, wer_ref, p_ref, featp_ref, er_ref):
    h = h_ref[...]
    featp_ref[...] = (
        jnp.dot(h, wf_ref[...], preferred_element_type=jnp.float32) + p_ref[...]
    )
    er_ref[...] = jnp.dot(h, wer_ref[...], preferred_element_type=jnp.float32)


def _tc_feat_call(h, wf, wer, p, block_rows=2000):
    n, k = h.shape
    wfw = wf.shape[1]
    return pl.pallas_call(
        _tc_feat_body,
        grid=(n // block_rows,),
        in_specs=[
            pl.BlockSpec((block_rows, k), lambda i: (i, 0)),
            pl.BlockSpec((k, wfw), lambda i: (0, 0)),
            pl.BlockSpec((k, 16), lambda i: (0, 0)),
            pl.BlockSpec((1, wfw), lambda i: (0, 0)),
        ],
        out_specs=[
            pl.BlockSpec((block_rows, wfw), lambda i: (i, 0)),
            pl.BlockSpec((block_rows, 16), lambda i: (i, 0)),
        ],
        out_shape=[
            jax.ShapeDtypeStruct((n, wfw), jnp.float32),
            jax.ShapeDtypeStruct((n, 16), jnp.float32),
        ],
    )(h, wf, wer, p)


def _tc_norm_feat_body(acc_ref, selw_ref, b_ref, wf_ref, wer_ref, p_ref,
                       featp_ref, er_ref):
    a = acc_ref[0] + acc_ref[1]
    den = a[:, 128:132]
    denb = jnp.dot(den, selw_ref[...], preferred_element_type=jnp.float32)
    h = a[:, :128] / (denb + _EPS) + b_ref[...]
    featp_ref[...] = (
        jnp.dot(h, wf_ref[...], preferred_element_type=jnp.float32) + p_ref[...]
    )
    er_ref[...] = jnp.dot(h, wer_ref[...], preferred_element_type=jnp.float32)


def _tc_norm_feat_call(acc, selw, brow, wf, wer, p, n, block_rows=2000):
    nc, _, wacc = acc.shape
    wfw = wf.shape[1]
    return pl.pallas_call(
        _tc_norm_feat_body,
        grid=(n // block_rows,),
        in_specs=[
            pl.BlockSpec((nc, block_rows, wacc), lambda i: (0, i, 0)),
            pl.BlockSpec((4, 128), lambda i: (0, 0)),
            pl.BlockSpec((1, 128), lambda i: (0, 0)),
            pl.BlockSpec((128, wfw), lambda i: (0, 0)),
            pl.BlockSpec((128, 16), lambda i: (0, 0)),
            pl.BlockSpec((1, wfw), lambda i: (0, 0)),
        ],
        out_specs=[
            pl.BlockSpec((block_rows, wfw), lambda i: (i, 0)),
            pl.BlockSpec((block_rows, 16), lambda i: (i, 0)),
        ],
        out_shape=[
            jax.ShapeDtypeStruct((n, wfw), jnp.float32),
            jax.ShapeDtypeStruct((n, 16), jnp.float32),
        ],
    )(acc, selw, brow, wf, wer, p)


def _tc_final_body(acc_ref, b_ref, out_ref):
    a = acc_ref[0] + acc_ref[1]
    den = a[:, 40:41]
    out_ref[...] = a[:, :40] / (den + _EPS) + b_ref[...]


def _tc_final_call(acc, brow, n, block_rows=2000):
    nc, _, wacc = acc.shape
    return pl.pallas_call(
        _tc_final_body,
        grid=(n // block_rows,),
        in_specs=[
            pl.BlockSpec((nc, block_rows, wacc), lambda i: (0, i, 0)),
            pl.BlockSpec((1, 40), lambda i: (0, 0)),
        ],
        out_specs=pl.BlockSpec((block_rows, 40), lambda i: (i, 0)),
        out_shape=jax.ShapeDtypeStruct((n, 40), jnp.float32),
    )(acc, brow)


@functools.lru_cache(maxsize=None)
def _make_sc_edge_kernel(n, e, wf, fcols, heads, b):
    nc, ns = 2, 16
    nw = nc * ns
    e_per_tile = e // nw
    iters = e_per_tile // b
    assert iters * b == e_per_tile and (iters - 2) % 3 == 0
    nv = wf // 16
    npad = ((n + 8 * ns - 1) // (8 * ns)) * (8 * ns)
    rows_per_tile = npad // ns
    sgroups = (b * heads) // 16
    mesh = plsc.VectorSubcoreMesh(
        core_axis_name="c", subcore_axis_name="s",
        num_cores=nc, num_subcores=ns)

    @functools.partial(
        pl.kernel,
        mesh=mesh,
        compiler_params=pltpu.CompilerParams(
            use_tc_tiling_on_sc=False, needs_layout_passes=False),
        out_type=jax.ShapeDtypeStruct((nc, npad, wf), jnp.float32),
        scratch_types=[
            pltpu.VMEM((3, b), jnp.int32),
            pltpu.VMEM((3, b), jnp.int32),
            pltpu.VMEM((3, b, wf), jnp.float32),
            pltpu.VMEM((3, b, 16), jnp.float32),
            pltpu.VMEM((b * heads + 16,), jnp.float32),
            pltpu.VMEM_SHARED((npad, wf), jnp.float32),
            [pltpu.SemaphoreType.DMA] * 3,
            [pltpu.SemaphoreType.DMA] * 3,
            [pltpu.SemaphoreType.DMA] * 3,
            [pltpu.SemaphoreType.DMA] * 3,
        ],
    )
    def sc_kernel(featp_hbm, er_hbm, src_hbm, dst_hbm, out_hbm,
                  srcv3, dstv3, fb3, eb3, sbuf, acc,
                  semi, semf, seme, semsc):
        cid = lax.axis_index("c")
        sid = lax.axis_index("s")
        wid = sid * nc + cid
        lane = lax.iota(jnp.int32, 16)
        zvec = jnp.zeros((16,), jnp.float32)
        last = iters - 1

        def zbody(r, _):
            for j in range(nv):
                fb3[0, r, pl.ds(j * 16, 16)] = zvec
            return 0

        lax.fori_loop(0, b, zbody, 0)
        row0 = sid * rows_per_tile
        zfull2 = rows_per_tile // b
        ztail2 = rows_per_tile - zfull2 * b
        assert ztail2 % 8 == 0
        for t in range(zfull2):
            pltpu.sync_copy(fb3.at[0], acc.at[pl.ds(row0 + t * b, b)])
        if ztail2:
            pltpu.sync_copy(
                fb3.at[0, pl.ds(0, ztail2)],
                acc.at[pl.ds(row0 + zfull2 * b, ztail2)])

        if heads == 4:
            s_rowp = lane >> 2
            s_colp_el = fcols + heads + (lane & 3)
            s_colp_er = lane & 3
        else:
            s_rowp = lane
            s_colp_el = jnp.full((16,), fcols + 1, jnp.int32)
            s_colp_er = jnp.zeros((16,), jnp.int32)
        tail_off = jnp.minimum(lane, heads - 1)

        def idx_start(k, bi):
            base = pl.multiple_of(wid * e_per_tile + k * b, 8)
            pltpu.async_copy(src_hbm.at[pl.ds(base, b)], srcv3.at[bi],
                             semi[bi])
            pltpu.async_copy(dst_hbm.at[pl.ds(base, b)], dstv3.at[bi],
                             semi[bi])

        def idx_wait(bi):
            pltpu.make_async_copy(
                src_hbm.at[pl.ds(0, b)], srcv3.at[bi], semi[bi]).wait()
            pltpu.make_async_copy(
                dst_hbm.at[pl.ds(0, b)], dstv3.at[bi], semi[bi]).wait()

        def gather_start(bi):
            pltpu.async_copy(featp_hbm.at[srcv3.at[bi]], fb3.at[bi],
                             semf[bi])
            pltpu.async_copy(er_hbm.at[dstv3.at[bi]], eb3.at[bi], seme[bi])

        def gather_wait(bi):
            pltpu.make_async_copy(
                featp_hbm.at[srcv3.at[bi]], fb3.at[bi], semf[bi]).wait()
            pltpu.make_async_copy(
                er_hbm.at[dstv3.at[bi]], eb3.at[bi], seme[bi]).wait()

        def scatter_start(bi):
            pltpu.async_copy(fb3.at[bi], acc.at[dstv3.at[bi]], semsc[bi],
                             add=True)

        def scatter_wait(bi):
            pltpu.make_async_copy(
                fb3.at[bi], acc.at[dstv3.at[bi]], semsc[bi]).wait()

        def compute(bi):
            fb = fb3.at[bi]
            eb = eb3.at[bi]

            def s_one(g):
                if heads == 4:
                    r = g * 4 + s_rowp
                else:
                    r = g * 16 + s_rowp
                el = plsc.load_gather(fb, [r, s_colp_el])
                er = plsc.load_gather(eb, [r, s_colp_er])
                v = el + er
                v = jnp.where(v >= 0.0, v, _NEG_SLOPE * v)
                sbuf[pl.ds(g * 16, 16)] = jnp.exp(v)

            def sbody(g, _):
                s_one(g)
                return 0

            lax.fori_loop(0, sgroups, sbody, 0)

            def mbody(k, _):
                kh = k * heads
                if heads == 4:
                    svs = [
                        plsc.load_gather(
                            sbuf, [jnp.full((16,), kh + h, jnp.int32)])
                        for h in range(4)
                    ]
                    stail = plsc.load_gather(sbuf, [kh + tail_off])
                    for j in range(nv):
                        sv = svs[j // 2] if j < 8 else stail
                        fb[k, pl.ds(j * 16, 16)] = (
                            fb[k, pl.ds(j * 16, 16)] * sv)
                else:
                    sv = plsc.load_gather(
                        sbuf, [jnp.full((16,), k, jnp.int32)])
                    for j in range(nv):
                        fb[k, pl.ds(j * 16, 16)] = (
                            fb[k, pl.ds(j * 16, 16)] * sv)
                return 0

            lax.fori_loop(0, b, mbody, 0)

        idx_start(0, 0)
        idx_start(1, 1)
        idx_start(2, 2)
        idx_wait(0)
        gather_start(0)
        idx_wait(1)
        gather_start(1)
        plsc.subcore_barrier()

        gather_wait(0)
        compute(0)
        idx_wait(2)
        gather_start(2)
        scatter_start(0)
        gather_wait(1)
        compute(1)
        scatter_wait(0)
        idx_start(3, 0)
        idx_wait(0)
        gather_start(0)
        scatter_start(1)

        def step(k, bi):
            gather_wait(bi)
            compute(bi)
            nb = (bi + 2) % 3
            scatter_wait(nb)
            idx_start(jnp.minimum(k + 2, last), nb)
            idx_wait(nb)
            gather_start(nb)
            scatter_start(bi)

        def pbody(g, _):
            k = 3 * g + 2
            step(k, 2)
            step(k + 1, 0)
            step(k + 2, 1)
            return 0

        lax.fori_loop(0, (iters - 2) // 3, pbody, 0)

        gather_wait(2)
        gather_wait(0)
        scatter_wait(1)

        plsc.subcore_barrier()
        pltpu.sync_copy(acc.at[pl.ds(row0, rows_per_tile)],
                        out_hbm.at[cid, pl.ds(row0, rows_per_tile)])

    return sc_kernel


def _prep_layer_weights(W, al, ar, heads, dout, wfw):
    k = W.shape[0]
    fcols = heads * dout
    wal = jnp.einsum("khd,hd->kh", W.reshape(k, heads, dout), al)
    war = jnp.einsum("khd,hd->kh", W.reshape(k, heads, dout), ar)
    wf = jnp.zeros((k, wfw), jnp.float32)
    wf = wf.at[:, :fcols].set(W)
    wf = wf.at[:, fcols + heads:fcols + 2 * heads].set(wal)
    wer = jnp.zeros((k, 16), jnp.float32)
    wer = wer.at[:, :heads].set(war)
    p = jnp.zeros((1, wfw), jnp.float32)
    p = p.at[0, fcols:fcols + heads].set(1.0)
    return wf, wer, p


def kernel(x, edge_index0, edge_index1, edge_index2,
           W0, al0, ar0, b0, W1, al1, ar1, b1, W2, al2, ar2, b2):
    n = x.shape[0]
    e = edge_index0.shape[1]

    wf0, wer0, p0 = _prep_layer_weights(W0, al0, ar0, 4, 32, 144)
    wf1, wer1, p1 = _prep_layer_weights(W1, al1, ar1, 4, 32, 144)
    wf2, wer2, p2 = _prep_layer_weights(W2, al2, ar2, 1, 40, 48)

    selw = jnp.zeros((4, 128), jnp.float32)
    for h in range(4):
        selw = selw.at[h, h * 32:(h + 1) * 32].set(1.0)
    b0row = b0.reshape(1, 128)
    b1row = b1.reshape(1, 128)
    b2row = b2.reshape(1, 40)

    bsz = 80
    ei0 = edge_index0.astype(jnp.int32)
    ei1 = edge_index1.astype(jnp.int32)
    ei2 = edge_index2.astype(jnp.int32)

    sc144 = _make_sc_edge_kernel(n, e, 144, 128, 4, bsz)
    sc48 = _make_sc_edge_kernel(n, e, 48, 40, 1, bsz)

    featp, er = _tc_feat_call(x, wf0, wer0, p0)
    acc0 = sc144(featp, er, ei0[0], ei0[1])

    featp, er = _tc_norm_feat_call(acc0, selw, b0row, wf1, wer1, p1, n)
    acc1 = sc144(featp, er, ei1[0], ei1[1])

    featp, er = _tc_norm_feat_call(acc1, selw, b1row, wf2, wer2, p2, n)
    acc2 = sc48(featp, er, ei2[0], ei2[1])

    return _tc_final_call(acc2, b2row, n)


# speedup vs baseline: 1.4324x; 1.3364x over previous
import functools

import jax
import jax.numpy as jnp
from jax import lax
from jax.experimental import pallas as pl
from jax.experimental.pallas import tpu as pltpu
from jax.experimental.pallas import tpu_sc as plsc

_NEG_SLOPE = 0.2
_EPS = 1e-9


def _tc_feat_body(h_ref, wf_ref, wer_ref, p_ref, featp_ref, er_ref):
    h = h_ref[...]
    featp_ref[...] = (
        jnp.dot(h, wf_ref[...], preferred_element_type=jnp.float32) + p_ref[...]
    )
    er_ref[...] = jnp.dot(h, wer_ref[...], preferred_element_type=jnp.float32)


def _tc_feat_call(h, wf, wer, p, block_rows=2000):
    n, k = h.shape
    wfw = wf.shape[1]
    return pl.pallas_call(
        _tc_feat_body,
        grid=(n // block_rows,),
        in_specs=[
            pl.BlockSpec((block_rows, k), lambda i: (i, 0)),
            pl.BlockSpec((k, wfw), lambda i: (0, 0)),
            pl.BlockSpec((k, 16), lambda i: (0, 0)),
            pl.BlockSpec((1, wfw), lambda i: (0, 0)),
        ],
        out_specs=[
            pl.BlockSpec((block_rows, wfw), lambda i: (i, 0)),
            pl.BlockSpec((block_rows, 16), lambda i: (i, 0)),
        ],
        out_shape=[
            jax.ShapeDtypeStruct((n, wfw), jnp.float32),
            jax.ShapeDtypeStruct((n, 16), jnp.float32),
        ],
    )(h, wf, wer, p)


def _tc_norm_feat_body(acc_ref, selw_ref, b_ref, wf_ref, wer_ref, p_ref,
                       featp_ref, er_ref):
    a = acc_ref[0] + acc_ref[1]
    den = a[:, 128:132]
    denb = jnp.dot(den, selw_ref[...], preferred_element_type=jnp.float32)
    h = a[:, :128] / (denb + _EPS) + b_ref[...]
    featp_ref[...] = (
        jnp.dot(h, wf_ref[...], preferred_element_type=jnp.float32) + p_ref[...]
    )
    er_ref[...] = jnp.dot(h, wer_ref[...], preferred_element_type=jnp.float32)


def _tc_norm_feat_call(acc, selw, brow, wf, wer, p, n, block_rows=2000):
    nc, _, wacc = acc.shape
    wfw = wf.shape[1]
    return pl.pallas_call(
        _tc_norm_feat_body,
        grid=(n // block_rows,),
        in_specs=[
            pl.BlockSpec((nc, block_rows, wacc), lambda i: (0, i, 0)),
            pl.BlockSpec((4, 128), lambda i: (0, 0)),
            pl.BlockSpec((1, 128), lambda i: (0, 0)),
            pl.BlockSpec((128, wfw), lambda i: (0, 0)),
            pl.BlockSpec((128, 16), lambda i: (0, 0)),
            pl.BlockSpec((1, wfw), lambda i: (0, 0)),
        ],
        out_specs=[
            pl.BlockSpec((block_rows, wfw), lambda i: (i, 0)),
            pl.BlockSpec((block_rows, 16), lambda i: (i, 0)),
        ],
        out_shape=[
            jax.ShapeDtypeStruct((n, wfw), jnp.float32),
            jax.ShapeDtypeStruct((n, 16), jnp.float32),
        ],
    )(acc, selw, brow, wf, wer, p)


def _tc_final_body(acc_ref, b_ref, out_ref):
    a = acc_ref[0] + acc_ref[1]
    den = a[:, 40:41]
    out_ref[...] = a[:, :40] / (den + _EPS) + b_ref[...]


def _tc_final_call(acc, brow, n, block_rows=2000):
    nc, _, wacc = acc.shape
    return pl.pallas_call(
        _tc_final_body,
        grid=(n // block_rows,),
        in_specs=[
            pl.BlockSpec((nc, block_rows, wacc), lambda i: (0, i, 0)),
            pl.BlockSpec((1, 40), lambda i: (0, 0)),
        ],
        out_specs=pl.BlockSpec((block_rows, 40), lambda i: (i, 0)),
        out_shape=jax.ShapeDtypeStruct((n, 40), jnp.float32),
    )(acc, brow)


@functools.lru_cache(maxsize=None)
def _make_sc_edge_kernel(n, e, wf, fcols, heads, b):
    nc, ns = 2, 16
    nw = nc * ns
    e_per_tile = e // nw
    iters = e_per_tile // b
    assert iters * b == e_per_tile and (iters - 2) % 3 == 0
    nv = wf // 16
    npad = ((n + 8 * ns - 1) // (8 * ns)) * (8 * ns)
    rows_per_tile = npad // ns
    sgroups = (b * heads) // 16
    mesh = plsc.VectorSubcoreMesh(
        core_axis_name="c", subcore_axis_name="s",
        num_cores=nc, num_subcores=ns)

    @functools.partial(
        pl.kernel,
        mesh=mesh,
        compiler_params=pltpu.CompilerParams(
            use_tc_tiling_on_sc=False, needs_layout_passes=False),
        out_type=jax.ShapeDtypeStruct((nc, npad, wf), jnp.float32),
        scratch_types=[
            pltpu.VMEM((3, b), jnp.int32),
            pltpu.VMEM((3, b), jnp.int32),
            pltpu.VMEM((3, b, wf), jnp.float32),
            pltpu.VMEM((3, b, 16), jnp.float32),
            pltpu.VMEM((b * heads + 16,), jnp.float32),
            pltpu.VMEM_SHARED((npad, wf), jnp.float32),
            [pltpu.SemaphoreType.DMA] * 3,
            [pltpu.SemaphoreType.DMA] * 3,
            [pltpu.SemaphoreType.DMA] * 3,
            [pltpu.SemaphoreType.DMA] * 3,
        ],
    )
    def sc_kernel(featp_hbm, er_hbm, src_hbm, dst_hbm, out_hbm,
                  srcv3, dstv3, fb3, eb3, sbuf, acc,
                  semi, semf, seme, semsc):
        cid = lax.axis_index("c")
        sid = lax.axis_index("s")
        wid = sid * nc + cid
        lane = lax.iota(jnp.int32, 16)
        zvec = jnp.zeros((16,), jnp.float32)
        last = iters - 1

        def zbody(r, _):
            for j in range(nv):
                fb3[0, r, pl.ds(j * 16, 16)] = zvec
            return 0

        lax.fori_loop(0, b, zbody, 0)
        row0 = sid * rows_per_tile
        zfull2 = rows_per_tile // b
        ztail2 = rows_per_tile - zfull2 * b
        assert ztail2 % 8 == 0
        for t in range(zfull2):
            pltpu.sync_copy(fb3.at[0], acc.at[pl.ds(row0 + t * b, b)])
        if ztail2:
            pltpu.sync_copy(
                fb3.at[0, pl.ds(0, ztail2)],
                acc.at[pl.ds(row0 + zfull2 * b, ztail2)])

        if heads == 4:
            s_rowp = lane >> 2
            s_colp_el = fcols + heads + (lane & 3)
            s_colp_er = lane & 3
        else:
            s_rowp = lane
            s_colp_el = jnp.full((16,), fcols + 1, jnp.int32)
            s_colp_er = jnp.zeros((16,), jnp.int32)
        tail_off = jnp.minimum(lane, heads - 1)

        def idx_start(k, bi):
            base = pl.multiple_of(wid * e_per_tile + k * b, 8)
            pltpu.async_copy(src_hbm.at[pl.ds(base, b)], srcv3.at[bi],
                             semi[bi])
            pltpu.async_copy(dst_hbm.at[pl.ds(base, b)], dstv3.at[bi],
                             semi[bi])

        def idx_wait(bi):
            pltpu.make_async_copy(
                src_hbm.at[pl.ds(0, b)], srcv3.at[bi], semi[bi]).wait()
            pltpu.make_async_copy(
                dst_hbm.at[pl.ds(0, b)], dstv3.at[bi], semi[bi]).wait()

        def gather_start(bi):
            pltpu.async_copy(featp_hbm.at[srcv3.at[bi]], fb3.at[bi],
                             semf[bi])
            pltpu.async_copy(er_hbm.at[dstv3.at[bi]], eb3.at[bi], seme[bi])

        def gather_wait(bi):
            pltpu.make_async_copy(
                featp_hbm.at[srcv3.at[bi]], fb3.at[bi], semf[bi]).wait()
            pltpu.make_async_copy(
                er_hbm.at[dstv3.at[bi]], eb3.at[bi], seme[bi]).wait()

        def scatter_start(bi):
            pltpu.async_copy(fb3.at[bi], acc.at[dstv3.at[bi]], semsc[bi],
                             add=True)

        def scatter_wait(bi):
            pltpu.make_async_copy(
                fb3.at[bi], acc.at[dstv3.at[bi]], semsc[bi]).wait()

        def compute(bi):
            fb = fb3.at[bi]
            eb = eb3.at[bi]

            def s_one(g):
                if heads == 4:
                    r = g * 4 + s_rowp
                else:
                    r = g * 16 + s_rowp
                el = plsc.load_gather(fb, [r, s_colp_el])
                er = plsc.load_gather(eb, [r, s_colp_er])
                v = el + er
                v = jnp.where(v >= 0.0, v, _NEG_SLOPE * v)
                sbuf[pl.ds(g * 16, 16)] = jnp.exp(v)

            def sbody(g, _):
                s_one(g)
                return 0

            lax.fori_loop(0, sgroups, sbody, 0)

            @functools.partial(plsc.parallel_loop, 0, b)
            def mbody(k):
                kh = k * heads
                if heads == 4:
                    svs = [
                        plsc.load_gather(
                            sbuf, [jnp.full((16,), kh + h, jnp.int32)])
                        for h in range(4)
                    ]
                    stail = plsc.load_gather(sbuf, [kh + tail_off])
                    for j in range(nv):
                        sv = svs[j // 2] if j < 8 else stail
                        fb[k, pl.ds(j * 16, 16)] = (
                            fb[k, pl.ds(j * 16, 16)] * sv)
                else:
                    sv = plsc.load_gather(
                        sbuf, [jnp.full((16,), k, jnp.int32)])
                    for j in range(nv):
                        fb[k, pl.ds(j * 16, 16)] = (
                            fb[k, pl.ds(j * 16, 16)] * sv)

        idx_start(0, 0)
        idx_start(1, 1)
        idx_start(2, 2)
        idx_wait(0)
        gather_start(0)
        idx_wait(1)
        gather_start(1)
        plsc.subcore_barrier()

        gather_wait(0)
        compute(0)
        idx_wait(2)
        gather_start(2)
        scatter_start(0)
        gather_wait(1)
        compute(1)
        scatter_wait(0)
        idx_start(3, 0)
        idx_wait(0)
        gather_start(0)
        scatter_start(1)

        def step(k, bi):
            gather_wait(bi)
            compute(bi)
            nb = (bi + 2) % 3
            scatter_wait(nb)
            idx_start(jnp.minimum(k + 2, last), nb)
            idx_wait(nb)
            gather_start(nb)
            scatter_start(bi)

        def pbody(g, _):
            k = 3 * g + 2
            step(k, 2)
            step(k + 1, 0)
            step(k + 2, 1)
            return 0

        lax.fori_loop(0, (iters - 2) // 3, pbody, 0)

        gather_wait(2)
        gather_wait(0)
        scatter_wait(1)

        plsc.subcore_barrier()
        pltpu.sync_copy(acc.at[pl.ds(row0, rows_per_tile)],
                        out_hbm.at[cid, pl.ds(row0, rows_per_tile)])

    return sc_kernel


def _prep_layer_weights(W, al, ar, heads, dout, wfw):
    k = W.shape[0]
    fcols = heads * dout
    wal = jnp.einsum("khd,hd->kh", W.reshape(k, heads, dout), al)
    war = jnp.einsum("khd,hd->kh", W.reshape(k, heads, dout), ar)
    wf = jnp.zeros((k, wfw), jnp.float32)
    wf = wf.at[:, :fcols].set(W)
    wf = wf.at[:, fcols + heads:fcols + 2 * heads].set(wal)
    wer = jnp.zeros((k, 16), jnp.float32)
    wer = wer.at[:, :heads].set(war)
    p = jnp.zeros((1, wfw), jnp.float32)
    p = p.at[0, fcols:fcols + heads].set(1.0)
    return wf, wer, p


def kernel(x, edge_index0, edge_index1, edge_index2,
           W0, al0, ar0, b0, W1, al1, ar1, b1, W2, al2, ar2, b2):
    n = x.shape[0]
    e = edge_index0.shape[1]

    wf0, wer0, p0 = _prep_layer_weights(W0, al0, ar0, 4, 32, 144)
    wf1, wer1, p1 = _prep_layer_weights(W1, al1, ar1, 4, 32, 144)
    wf2, wer2, p2 = _prep_layer_weights(W2, al2, ar2, 1, 40, 48)

    selw = jnp.zeros((4, 128), jnp.float32)
    for h in range(4):
        selw = selw.at[h, h * 32:(h + 1) * 32].set(1.0)
    b0row = b0.reshape(1, 128)
    b1row = b1.reshape(1, 128)
    b2row = b2.reshape(1, 40)

    bsz = 80
    ei0 = edge_index0.astype(jnp.int32)
    ei1 = edge_index1.astype(jnp.int32)
    ei2 = edge_index2.astype(jnp.int32)

    sc144 = _make_sc_edge_kernel(n, e, 144, 128, 4, bsz)
    sc48 = _make_sc_edge_kernel(n, e, 48, 40, 1, bsz)

    featp, er = _tc_feat_call(x, wf0, wer0, p0)
    acc0 = sc144(featp, er, ei0[0], ei0[1])

    featp, er = _tc_norm_feat_call(acc0, selw, b0row, wf1, wer1, p1, n)
    acc1 = sc144(featp, er, ei1[0], ei1[1])

    featp, er = _tc_norm_feat_call(acc1, selw, b1row, wf2, wer2, p2, n)
    acc2 = sc48(featp, er, ei2[0], ei2[1])

    return _tc_final_call(acc2, b2row, n)
